# baseline (speedup 1.0000x reference)
import functools

import jax
import jax.numpy as jnp
from jax.experimental import pallas as pl
from jax.experimental.pallas import tpu as pltpu

_IN = 784
_H1 = 256
_H2 = 32


def _fused_body(x_ref, eps_ref,
                w1_ref, b1_ref, w2_ref, b2_ref, wh_ref, bh_ref,
                v1_ref, c1_ref, v2_ref, c2_ref, v3_ref, c3_ref,
                xhat_ref, mu_ref, lv_ref):
    zd = mu_ref.shape[-1]

    xb = x_ref[...].astype(jnp.bfloat16)
    a = jnp.dot(xb, w1_ref[...], preferred_element_type=jnp.float32)
    a = jnp.maximum(a + b1_ref[...], 0.0).astype(jnp.bfloat16)
    a = jnp.dot(a, w2_ref[...], preferred_element_type=jnp.float32)
    a = jnp.maximum(a + b2_ref[...], 0.0).astype(jnp.bfloat16)

    heads = jnp.dot(a, wh_ref[...], preferred_element_type=jnp.float32)
    heads = heads + bh_ref[...]
    mu = heads[:, :zd]
    lv = heads[:, zd:]
    mu_ref[...] = mu
    lv_ref[...] = lv

    z = (mu + eps_ref[...] * jnp.exp(0.5 * lv)).astype(jnp.bfloat16)
    g = jnp.dot(z, v1_ref[...], preferred_element_type=jnp.float32)
    g = jnp.maximum(g + c1_ref[...], 0.0).astype(jnp.bfloat16)
    g = jnp.dot(g, v2_ref[...], preferred_element_type=jnp.float32)
    g = jnp.maximum(g + c2_ref[...], 0.0).astype(jnp.bfloat16)
    g = jnp.dot(g, v3_ref[...], preferred_element_type=jnp.float32)
    xhat_ref[...] = jax.nn.sigmoid(g + c3_ref[...])


@jax.jit
def _vae_forward(e_w1, e_b1, e_w2, e_b2, e_wh, e_bh,
                 d_w1, d_b1, d_w2, d_b2, d_w3, d_b3, x, eps):
    B = x.shape[0]
    zd = eps.shape[1]
    x2 = x.reshape(B, _IN)

    TB = B if B <= 512 else 512
    n_tiles = pl.cdiv(B, TB)
    Bp = n_tiles * TB
    if Bp != B:
        x2 = jnp.pad(x2, ((0, Bp - B), (0, 0)))
        eps = jnp.pad(eps, ((0, Bp - B), (0, 0)))

    wts = (e_w1.astype(jnp.bfloat16), e_b1,
           e_w2.astype(jnp.bfloat16), e_b2,
           e_wh.astype(jnp.bfloat16), e_bh,
           d_w1.astype(jnp.bfloat16), d_b1,
           d_w2.astype(jnp.bfloat16), d_b2,
           d_w3.astype(jnp.bfloat16), d_b3)

    def tile_spec(f):
        return pl.BlockSpec((TB, f), lambda i: (i, 0))

    def pinned(a):
        return pl.BlockSpec(a.shape, lambda i: (0, 0))

    flops = 2 * Bp * (_IN * _H1 + _H1 * _H2 + _H2 * 2 * zd
                      + zd * _H2 + _H2 * _H1 + _H1 * _IN)
    cost = pl.CostEstimate(
        flops=flops,
        transcendentals=Bp * (zd + _IN),
        bytes_accessed=4 * Bp * (2 * _IN + 3 * zd) + 2 * sum(int(w.size) for w in wts),
    )

    xhat, mu, lv = pl.pallas_call(
        _fused_body,
        out_shape=(
            jax.ShapeDtypeStruct((Bp, _IN), jnp.float32),
            jax.ShapeDtypeStruct((Bp, zd), jnp.float32),
            jax.ShapeDtypeStruct((Bp, zd), jnp.float32),
        ),
        grid=(n_tiles,),
        in_specs=[tile_spec(_IN), tile_spec(zd)] + [pinned(w) for w in wts],
        out_specs=(tile_spec(_IN), tile_spec(zd), tile_spec(zd)),
        compiler_params=pltpu.CompilerParams(
            dimension_semantics=("parallel",),
            vmem_limit_bytes=64 * 1024 * 1024,
        ),
        cost_estimate=cost,
    )(x2, eps, *wts)

    return xhat[:B].reshape(B, 1, 28, 28), mu[:B], lv[:B]


def kernel(e_w1, e_b1, e_w2, e_b2, e_wh, e_bh,
           d_w1, d_b1, d_w2, d_b2, d_w3, d_b3, x, eps):
    return _vae_forward(e_w1, e_b1, e_w2, e_b2, e_wh, e_bh,
                        d_w1, d_b1, d_w2, d_b2, d_w3, d_b3, x, eps)
